# baseline (speedup 1.0000x reference)
import functools

import numpy as np
import jax
import jax.numpy as jnp
from jax import lax
from jax.experimental import pallas as pl
from jax.experimental.pallas import tpu as pltpu
from jax.experimental.pallas import tpu_sc as plsc

_SEQ = 200
_D = 64
_B = 1024
_NC, _NS = 2, 16
_NW = _NC * _NS
_SEQ_PER_W = _B // _NW
_CHUNK = 100
_CPS = _SEQ // _CHUNK
_IDX_ROWS_PER_W = _SEQ_PER_W * _CPS


def _pos_encoding():
    pos = np.arange(_SEQ)[:, np.newaxis]
    i = np.arange(_D)[np.newaxis, :]
    angle_rates = 1.0 / np.power(10000, 2 * (i // 2) / np.float32(_D))
    angle_rads = pos * angle_rates
    angle_rads[:, 0::2] = np.sin(angle_rads[:, 0::2])
    angle_rads[:, 1::2] = np.cos(angle_rads[:, 1::2])
    return np.asarray(angle_rads, dtype=np.float32)


def _embed_sc(table, idx2d, pos):
    mesh = plsc.VectorSubcoreMesh(
        core_axis_name="c", subcore_axis_name="s",
        num_cores=_NC, num_subcores=_NS,
    )

    @functools.partial(
        pl.kernel,
        out_type=jax.ShapeDtypeStruct((_B, _SEQ, _D), jnp.float32),
        mesh=mesh,
        scratch_types=[
            pltpu.VMEM((_IDX_ROWS_PER_W, _CHUNK), jnp.int32),
            pltpu.VMEM((_SEQ, _D), jnp.float32),
            pltpu.VMEM((_SEQ, _D), jnp.float32),
            pltpu.SemaphoreType.DMA,
        ],
        compiler_params=pltpu.CompilerParams(use_tc_tiling_on_sc=False),
    )
    def k(table_hbm, idx_hbm, pos_hbm, out_hbm, idx_v, pos_v, buf, sem):
        wid = lax.axis_index("s") * _NC + lax.axis_index("c")
        pltpu.sync_copy(idx_hbm.at[pl.ds(wid * _IDX_ROWS_PER_W, _IDX_ROWS_PER_W)], idx_v)
        pltpu.sync_copy(pos_hbm, pos_v)

        def seq_body(s, carry):
            c0 = pltpu.async_copy(
                table_hbm.at[idx_v.at[_CPS * s]], buf.at[pl.ds(0, _CHUNK)], sem)
            c1 = pltpu.async_copy(
                table_hbm.at[idx_v.at[_CPS * s + 1]], buf.at[pl.ds(_CHUNK, _CHUNK)], sem)
            c0.wait()
            c1.wait()

            def row_body(l, c2):
                for j in range(_D // 16):
                    v = buf[l, pl.ds(j * 16, 16)]
                    p = pos_v[l, pl.ds(j * 16, 16)]
                    buf[l, pl.ds(j * 16, 16)] = v * 8.0 + p
                return c2

            lax.fori_loop(0, _SEQ, row_body, 0)
            pltpu.sync_copy(buf, out_hbm.at[wid * _SEQ_PER_W + s])
            return carry

        lax.fori_loop(0, _SEQ_PER_W, seq_body, 0)

    return k(table, idx2d, pos)


def kernel(inputs, table):
    pos = jnp.asarray(_pos_encoding())
    idx2d = inputs.reshape(_NW * _IDX_ROWS_PER_W, _CHUNK)
    return _embed_sc(table, idx2d, pos)


# speedup vs baseline: 1.2294x; 1.2294x over previous
import functools

import numpy as np
import jax
import jax.numpy as jnp
from jax import lax
from jax.experimental import pallas as pl
from jax.experimental.pallas import tpu as pltpu
from jax.experimental.pallas import tpu_sc as plsc

_SEQ = 200
_D = 64
_B = 1024
_V = 1000000
_NC, _NS = 2, 16
_NW = _NC * _NS
_SEQ_PER_W = _B // _NW
_CHUNK = 100
_CPS = _SEQ // _CHUNK
_IDX_ROWS_PER_W = _SEQ_PER_W * _CPS

_FMT_BLK = 4096


def _pos_encoding():
    pos = np.arange(_SEQ)[:, np.newaxis]
    i = np.arange(_D)[np.newaxis, :]
    angle_rates = 1.0 / np.power(10000, 2 * (i // 2) / np.float32(_D))
    angle_rads = pos * angle_rates
    angle_rads[:, 0::2] = np.sin(angle_rads[:, 0::2])
    angle_rads[:, 1::2] = np.cos(angle_rads[:, 1::2])
    return np.asarray(angle_rads, dtype=np.float32)


def _fmt_body(tabt_ref, out_ref):
    x = tabt_ref[...]
    out_ref[:, 0:_D] = jnp.swapaxes(x, 0, 1) * 8.0


def _format_tc(tab_t):
    grid = (_V + _FMT_BLK - 1) // _FMT_BLK
    return pl.pallas_call(
        _fmt_body,
        grid=(grid,),
        in_specs=[pl.BlockSpec((_D, _FMT_BLK), lambda i: (0, i))],
        out_specs=pl.BlockSpec((_FMT_BLK, 2 * _D), lambda i: (i, 0)),
        out_shape=jax.ShapeDtypeStruct((_V, 2 * _D), jnp.float32),
    )(tab_t)


def _embed_sc(table, idx2d, pos):
    mesh = plsc.VectorSubcoreMesh(
        core_axis_name="c", subcore_axis_name="s",
        num_cores=_NC, num_subcores=_NS,
    )

    @functools.partial(
        pl.kernel,
        out_type=jax.ShapeDtypeStruct((_B, _SEQ, _D), jnp.float32),
        mesh=mesh,
        scratch_types=[
            pltpu.VMEM((_IDX_ROWS_PER_W, _CHUNK), jnp.int32),
            pltpu.VMEM((_SEQ, _D), jnp.float32),
            pltpu.VMEM((_SEQ, 2 * _D), jnp.float32),
            pltpu.VMEM((_SEQ, _D), jnp.float32),
            pltpu.SemaphoreType.DMA,
        ],
        compiler_params=pltpu.CompilerParams(use_tc_tiling_on_sc=False),
    )
    def k(table_hbm, idx_hbm, pos_hbm, out_hbm, idx_v, pos_v, buf, obuf, sem):
        wid = lax.axis_index("s") * _NC + lax.axis_index("c")
        pltpu.sync_copy(idx_hbm.at[pl.ds(wid * _IDX_ROWS_PER_W, _IDX_ROWS_PER_W)], idx_v)
        pltpu.sync_copy(pos_hbm, pos_v)

        def seq_body(s, carry):
            c0 = pltpu.async_copy(
                table_hbm.at[idx_v.at[_CPS * s]], buf.at[pl.ds(0, _CHUNK)], sem)
            c1 = pltpu.async_copy(
                table_hbm.at[idx_v.at[_CPS * s + 1]], buf.at[pl.ds(_CHUNK, _CHUNK)], sem)
            c0.wait()
            c1.wait()

            def row_body(l, c2):
                for j in range(_D // 16):
                    v = buf[l, pl.ds(j * 16, 16)]
                    p = pos_v[l, pl.ds(j * 16, 16)]
                    obuf[l, pl.ds(j * 16, 16)] = v + p
                return c2

            lax.fori_loop(0, _SEQ, row_body, 0)
            pltpu.sync_copy(obuf, out_hbm.at[wid * _SEQ_PER_W + s])
            return carry

        lax.fori_loop(0, _SEQ_PER_W, seq_body, 0)

    return k(table, idx2d, pos)


def kernel(inputs, table):
    tab_t = jnp.transpose(table)
    tab2 = _format_tc(tab_t)
    pos = jnp.asarray(_pos_encoding())
    idx2d = inputs.reshape(_NW * _IDX_ROWS_PER_W, _CHUNK)
    return _embed_sc(tab2, idx2d, pos)


# speedup vs baseline: 1.4530x; 1.1819x over previous
import functools

import numpy as np
import jax
import jax.numpy as jnp
from jax import lax
from jax.experimental import pallas as pl
from jax.experimental.pallas import tpu as pltpu
from jax.experimental.pallas import tpu_sc as plsc

_SEQ = 200
_D = 64
_B = 1024
_V = 1000000
_NC, _NS = 2, 16
_NW = _NC * _NS
_SEQ_PER_W = _B // _NW
_CHUNK = 100
_CPS = _SEQ // _CHUNK
_IDX_ROWS_PER_W = _SEQ_PER_W * _CPS

_FMT_BLK = 4096


def _pos_encoding():
    pos = np.arange(_SEQ)[:, np.newaxis]
    i = np.arange(_D)[np.newaxis, :]
    angle_rates = 1.0 / np.power(10000, 2 * (i // 2) / np.float32(_D))
    angle_rads = pos * angle_rates
    angle_rads[:, 0::2] = np.sin(angle_rads[:, 0::2])
    angle_rads[:, 1::2] = np.cos(angle_rads[:, 1::2])
    return np.asarray(angle_rads, dtype=np.float32)


def _fmt_body(tabt_ref, out_ref):
    x = tabt_ref[...]
    out_ref[:, 0:_D] = jnp.swapaxes(x, 0, 1) * 8.0


def _format_tc(tab_t):
    grid = (_V + _FMT_BLK - 1) // _FMT_BLK
    return pl.pallas_call(
        _fmt_body,
        grid=(grid,),
        in_specs=[pl.BlockSpec((_D, _FMT_BLK), lambda i: (0, i))],
        out_specs=pl.BlockSpec((_FMT_BLK, 2 * _D), lambda i: (i, 0)),
        out_shape=jax.ShapeDtypeStruct((_V, 2 * _D), jnp.float32),
    )(tab_t)


def _embed_sc(table, idx2d, pos):
    mesh = plsc.VectorSubcoreMesh(
        core_axis_name="c", subcore_axis_name="s",
        num_cores=_NC, num_subcores=_NS,
    )

    @functools.partial(
        pl.kernel,
        out_type=jax.ShapeDtypeStruct((_B, _SEQ, _D), jnp.float32),
        mesh=mesh,
        scratch_types=[
            pltpu.VMEM((_IDX_ROWS_PER_W, _CHUNK), jnp.int32),
            pltpu.VMEM((_SEQ, _D), jnp.float32),
            pltpu.VMEM((_SEQ, _D), jnp.float32),
            pltpu.SemaphoreType.DMA,
        ],
        compiler_params=pltpu.CompilerParams(use_tc_tiling_on_sc=False),
    )
    def k(table_hbm, idx_hbm, pos_hbm, out_hbm, idx_v, pos_v, buf, sem):
        wid = lax.axis_index("s") * _NC + lax.axis_index("c")
        pltpu.sync_copy(idx_hbm.at[pl.ds(wid * _IDX_ROWS_PER_W, _IDX_ROWS_PER_W)], idx_v)
        pltpu.sync_copy(pos_hbm, pos_v)

        def seq_body(s, carry):
            c0 = pltpu.async_copy(
                table_hbm.at[idx_v.at[_CPS * s]], buf.at[pl.ds(0, _CHUNK)], sem)
            c1 = pltpu.async_copy(
                table_hbm.at[idx_v.at[_CPS * s + 1]], buf.at[pl.ds(_CHUNK, _CHUNK)], sem)
            c0.wait()
            c1.wait()

            def row_body(l, c2):
                for j in range(_D // 16):
                    v = buf[l, pl.ds(j * 16, 16)]
                    p = pos_v[l, pl.ds(j * 16, 16)]
                    buf[l, pl.ds(j * 16, 16)] = v + p
                return c2

            lax.fori_loop(0, _SEQ, row_body, 0)
            pltpu.sync_copy(buf, out_hbm.at[wid * _SEQ_PER_W + s])
            return carry

        lax.fori_loop(0, _SEQ_PER_W, seq_body, 0)

    return k(table, idx2d, pos)


def kernel(inputs, table):
    tab_t = jnp.transpose(table)
    tab2 = _format_tc(tab_t)
    tab3 = tab2.reshape(2 * _V, _D)
    pos = jnp.asarray(_pos_encoding())
    idx2d = (inputs * 2).reshape(_NW * _IDX_ROWS_PER_W, _CHUNK)
    return _embed_sc(tab3, idx2d, pos)


# speedup vs baseline: 1.4914x; 1.0264x over previous
import functools

import numpy as np
import jax
import jax.numpy as jnp
from jax import lax
from jax.experimental import pallas as pl
from jax.experimental.pallas import tpu as pltpu
from jax.experimental.pallas import tpu_sc as plsc

_SEQ = 200
_D = 64
_B = 1024
_V = 1000000
_NC, _NS = 2, 16
_NW = _NC * _NS
_SEQ_PER_W = _B // _NW
_CHUNK = 100
_CPS = _SEQ // _CHUNK
_IDX_ROWS_PER_W = _SEQ_PER_W * _CPS

_FMT_BLK = 2048
_FMT_GRID = 245
_SPLIT = _FMT_BLK * _FMT_GRID


def _pos_encoding():
    pos = np.arange(_SEQ)[:, np.newaxis]
    i = np.arange(_D)[np.newaxis, :]
    angle_rates = 1.0 / np.power(10000, 2 * (i // 2) / np.float32(_D))
    angle_rads = pos * angle_rates
    angle_rads[:, 0::2] = np.sin(angle_rads[:, 0::2])
    angle_rads[:, 1::2] = np.cos(angle_rads[:, 1::2])
    return np.asarray(angle_rads, dtype=np.float32)


def _fmt_body(lo_ref, hi_ref, out_ref):
    out_ref[:, 0:_D] = jnp.swapaxes(lo_ref[...], 0, 1) * 8.0
    out_ref[:, _D:2 * _D] = jnp.swapaxes(hi_ref[...], 0, 1) * 8.0


def _format_tc(tab_t):
    return pl.pallas_call(
        _fmt_body,
        grid=(_FMT_GRID,),
        in_specs=[
            pl.BlockSpec((_D, _FMT_BLK), lambda i: (0, i)),
            pl.BlockSpec(
                (_D, _FMT_BLK),
                lambda i: (0, jnp.minimum(_FMT_GRID + i, _V // _FMT_BLK)),
            ),
        ],
        out_specs=pl.BlockSpec((_FMT_BLK, 2 * _D), lambda i: (i, 0)),
        out_shape=jax.ShapeDtypeStruct((_SPLIT, 2 * _D), jnp.float32),
    )(tab_t, tab_t)


def _embed_sc(table, idx2d, pos):
    mesh = plsc.VectorSubcoreMesh(
        core_axis_name="c", subcore_axis_name="s",
        num_cores=_NC, num_subcores=_NS,
    )

    @functools.partial(
        pl.kernel,
        out_type=jax.ShapeDtypeStruct((_B, _SEQ, _D), jnp.float32),
        mesh=mesh,
        scratch_types=[
            pltpu.VMEM((_IDX_ROWS_PER_W, _CHUNK), jnp.int32),
            pltpu.VMEM((_SEQ, _D), jnp.float32),
            pltpu.VMEM((_SEQ, _D), jnp.float32),
            pltpu.SemaphoreType.DMA,
        ],
        compiler_params=pltpu.CompilerParams(use_tc_tiling_on_sc=False),
    )
    def k(table_hbm, idx_hbm, pos_hbm, out_hbm, idx_v, pos_v, buf, sem):
        wid = lax.axis_index("s") * _NC + lax.axis_index("c")
        pltpu.sync_copy(idx_hbm.at[pl.ds(wid * _IDX_ROWS_PER_W, _IDX_ROWS_PER_W)], idx_v)
        pltpu.sync_copy(pos_hbm, pos_v)

        def seq_body(s, carry):
            c0 = pltpu.async_copy(
                table_hbm.at[idx_v.at[_CPS * s]], buf.at[pl.ds(0, _CHUNK)], sem)
            c1 = pltpu.async_copy(
                table_hbm.at[idx_v.at[_CPS * s + 1]], buf.at[pl.ds(_CHUNK, _CHUNK)], sem)
            c0.wait()
            c1.wait()

            def row_body(l, c2):
                for j in range(_D // 16):
                    v = buf[l, pl.ds(j * 16, 16)]
                    p = pos_v[l, pl.ds(j * 16, 16)]
                    buf[l, pl.ds(j * 16, 16)] = v + p
                return c2

            lax.fori_loop(0, _SEQ, row_body, 0)
            pltpu.sync_copy(buf, out_hbm.at[wid * _SEQ_PER_W + s])
            return carry

        lax.fori_loop(0, _SEQ_PER_W, seq_body, 0)

    return k(table, idx2d, pos)


def kernel(inputs, table):
    tab_t = jnp.transpose(table)
    tab2 = _format_tc(tab_t)
    tab3 = tab2.reshape(2 * _SPLIT, _D)
    pos = jnp.asarray(_pos_encoding())
    idx = jnp.where(inputs < _SPLIT, 2 * inputs, 2 * (inputs - _SPLIT) + 1)
    idx2d = idx.reshape(_NW * _IDX_ROWS_PER_W, _CHUNK)
    return _embed_sc(tab3, idx2d, pos)


# speedup vs baseline: 1.8128x; 1.2155x over previous
import functools

import numpy as np
import jax
import jax.numpy as jnp
from jax import lax
from jax.experimental import pallas as pl
from jax.experimental.pallas import tpu as pltpu
from jax.experimental.pallas import tpu_sc as plsc

_SEQ = 200
_D = 64
_B = 1024
_V = 1000000
_NC, _NS = 2, 16
_NW = _NC * _NS
_SEQ_PER_W = _B // _NW
_CHUNK = 100
_CPS = _SEQ // _CHUNK
_IDX_ROWS_PER_W = _SEQ_PER_W * _CPS

_FMT_BLK = 8192
_FMT_GRID = 62
_SPLIT = _FMT_BLK * _FMT_GRID


def _pos_encoding():
    pos = np.arange(_SEQ)[:, np.newaxis]
    i = np.arange(_D)[np.newaxis, :]
    angle_rates = 1.0 / np.power(10000, 2 * (i // 2) / np.float32(_D))
    angle_rads = pos * angle_rates
    angle_rads[:, 0::2] = np.sin(angle_rads[:, 0::2])
    angle_rads[:, 1::2] = np.cos(angle_rads[:, 1::2])
    return np.asarray(angle_rads, dtype=np.float32)


def _fmt_body(lo_ref, hi_ref, out_ref):
    out_ref[:, 0:_D] = jnp.swapaxes(lo_ref[...], 0, 1) * 8.0
    out_ref[:, _D:2 * _D] = jnp.swapaxes(hi_ref[...], 0, 1) * 8.0


def _format_tc(tab_t):
    return pl.pallas_call(
        _fmt_body,
        grid=(_FMT_GRID,),
        in_specs=[
            pl.BlockSpec((_D, _FMT_BLK), lambda i: (0, i)),
            pl.BlockSpec(
                (_D, _FMT_BLK),
                lambda i: (0, jnp.minimum(_FMT_GRID + i, _V // _FMT_BLK)),
            ),
        ],
        out_specs=pl.BlockSpec((_FMT_BLK, 2 * _D), lambda i: (i, 0)),
        out_shape=jax.ShapeDtypeStruct((_SPLIT, 2 * _D), jnp.float32),
    )(tab_t, tab_t)


def _embed_sc(table, idx2d, pos):
    mesh = plsc.VectorSubcoreMesh(
        core_axis_name="c", subcore_axis_name="s",
        num_cores=_NC, num_subcores=_NS,
    )

    @functools.partial(
        pl.kernel,
        out_type=jax.ShapeDtypeStruct((_B, _SEQ, _D), jnp.float32),
        mesh=mesh,
        scratch_types=[
            pltpu.VMEM((_IDX_ROWS_PER_W, _CHUNK), jnp.int32),
            pltpu.VMEM((_SEQ, _D), jnp.float32),
            pltpu.VMEM((_SEQ, _D), jnp.float32),
            pltpu.SemaphoreType.DMA,
        ],
        compiler_params=pltpu.CompilerParams(use_tc_tiling_on_sc=False),
    )
    def k(table_hbm, idx_hbm, pos_hbm, out_hbm, idx_v, pos_v, buf, sem):
        wid = lax.axis_index("s") * _NC + lax.axis_index("c")
        pltpu.sync_copy(idx_hbm.at[pl.ds(wid * _IDX_ROWS_PER_W, _IDX_ROWS_PER_W)], idx_v)
        pltpu.sync_copy(pos_hbm, pos_v)

        def seq_body(s, carry):
            c0 = pltpu.async_copy(
                table_hbm.at[idx_v.at[_CPS * s]], buf.at[pl.ds(0, _CHUNK)], sem)
            c1 = pltpu.async_copy(
                table_hbm.at[idx_v.at[_CPS * s + 1]], buf.at[pl.ds(_CHUNK, _CHUNK)], sem)
            c0.wait()
            c1.wait()

            def row_body(l, c2):
                for j in range(_D // 16):
                    v = buf[l, pl.ds(j * 16, 16)]
                    p = pos_v[l, pl.ds(j * 16, 16)]
                    buf[l, pl.ds(j * 16, 16)] = v + p
                return c2

            lax.fori_loop(0, _SEQ, row_body, 0)
            pltpu.sync_copy(buf, out_hbm.at[wid * _SEQ_PER_W + s])
            return carry

        lax.fori_loop(0, _SEQ_PER_W, seq_body, 0)

    return k(table, idx2d, pos)


def kernel(inputs, table):
    tab_t = jnp.transpose(table)
    tab2 = _format_tc(tab_t)
    tab3 = tab2.reshape(2 * _SPLIT, _D)
    pos = jnp.asarray(_pos_encoding())
    idx = jnp.where(inputs < _SPLIT, 2 * inputs, 2 * (inputs - _SPLIT) + 1)
    idx2d = idx.reshape(_NW * _IDX_ROWS_PER_W, _CHUNK)
    return _embed_sc(tab3, idx2d, pos)


# speedup vs baseline: 1.8714x; 1.0323x over previous
import functools

import numpy as np
import jax
import jax.numpy as jnp
from jax import lax
from jax.experimental import pallas as pl
from jax.experimental.pallas import tpu as pltpu
from jax.experimental.pallas import tpu_sc as plsc

_SEQ = 200
_D = 64
_B = 1024
_V = 1000000
_NC, _NS = 2, 16
_NW = _NC * _NS
_SEQ_PER_W = _B // _NW
_CHUNK = 100
_CPS = _SEQ // _CHUNK
_IDX_ROWS_PER_W = _SEQ_PER_W * _CPS

_FMT_BLK = 16384
_FMT_GRID = 31
_SPLIT = _FMT_BLK * _FMT_GRID


def _pos_encoding():
    pos = np.arange(_SEQ)[:, np.newaxis]
    i = np.arange(_D)[np.newaxis, :]
    angle_rates = 1.0 / np.power(10000, 2 * (i // 2) / np.float32(_D))
    angle_rads = pos * angle_rates
    angle_rads[:, 0::2] = np.sin(angle_rads[:, 0::2])
    angle_rads[:, 1::2] = np.cos(angle_rads[:, 1::2])
    return np.asarray(angle_rads, dtype=np.float32)


def _fmt_body(lo_ref, hi_ref, out_ref):
    out_ref[:, 0:_D] = jnp.swapaxes(lo_ref[...], 0, 1) * 8.0
    out_ref[:, _D:2 * _D] = jnp.swapaxes(hi_ref[...], 0, 1) * 8.0


def _format_tc(tab_t):
    return pl.pallas_call(
        _fmt_body,
        grid=(_FMT_GRID,),
        in_specs=[
            pl.BlockSpec((_D, _FMT_BLK), lambda i: (0, i)),
            pl.BlockSpec(
                (_D, _FMT_BLK),
                lambda i: (0, jnp.minimum(_FMT_GRID + i, _V // _FMT_BLK)),
            ),
        ],
        out_specs=pl.BlockSpec((_FMT_BLK, 2 * _D), lambda i: (i, 0)),
        out_shape=jax.ShapeDtypeStruct((_SPLIT, 2 * _D), jnp.float32),
    )(tab_t, tab_t)


def _embed_sc(table, idx2d, pos):
    mesh = plsc.VectorSubcoreMesh(
        core_axis_name="c", subcore_axis_name="s",
        num_cores=_NC, num_subcores=_NS,
    )

    @functools.partial(
        pl.kernel,
        out_type=jax.ShapeDtypeStruct((_B, _SEQ, _D), jnp.float32),
        mesh=mesh,
        scratch_types=[
            pltpu.VMEM((_IDX_ROWS_PER_W, _CHUNK), jnp.int32),
            pltpu.VMEM((_SEQ, _D), jnp.float32),
            pltpu.VMEM((_SEQ, _D), jnp.float32),
            pltpu.SemaphoreType.DMA,
        ],
        compiler_params=pltpu.CompilerParams(use_tc_tiling_on_sc=False),
    )
    def k(table_hbm, idx_hbm, pos_hbm, out_hbm, idx_v, pos_v, buf, sem):
        wid = lax.axis_index("s") * _NC + lax.axis_index("c")
        pltpu.sync_copy(idx_hbm.at[pl.ds(wid * _IDX_ROWS_PER_W, _IDX_ROWS_PER_W)], idx_v)
        pltpu.sync_copy(pos_hbm, pos_v)

        def seq_body(s, carry):
            c0 = pltpu.async_copy(
                table_hbm.at[idx_v.at[_CPS * s]], buf.at[pl.ds(0, _CHUNK)], sem)
            c1 = pltpu.async_copy(
                table_hbm.at[idx_v.at[_CPS * s + 1]], buf.at[pl.ds(_CHUNK, _CHUNK)], sem)
            c0.wait()
            c1.wait()

            def row_body(l, c2):
                for j in range(_D // 16):
                    v = buf[l, pl.ds(j * 16, 16)]
                    p = pos_v[l, pl.ds(j * 16, 16)]
                    buf[l, pl.ds(j * 16, 16)] = v + p
                return c2

            lax.fori_loop(0, _SEQ, row_body, 0)
            pltpu.sync_copy(buf, out_hbm.at[wid * _SEQ_PER_W + s])
            return carry

        lax.fori_loop(0, _SEQ_PER_W, seq_body, 0)

    return k(table, idx2d, pos)


def kernel(inputs, table):
    tab_t = jnp.transpose(table)
    tab2 = _format_tc(tab_t)
    tab3 = tab2.reshape(2 * _SPLIT, _D)
    pos = jnp.asarray(_pos_encoding())
    idx = jnp.where(inputs < _SPLIT, 2 * inputs, 2 * (inputs - _SPLIT) + 1)
    idx2d = idx.reshape(_NW * _IDX_ROWS_PER_W, _CHUNK)
    return _embed_sc(tab3, idx2d, pos)
